# Initial kernel scaffold; baseline (speedup 1.0000x reference)
import jax
import jax.numpy as jnp
from jax.experimental import pallas as pl


def kernel(x, Wg, W1, W2, W3):
    raise NotImplementedError("write your pallas kernel here")


# speedup vs baseline: 1.9587x; 1.9587x over previous
import functools

import jax
import jax.numpy as jnp
from jax import lax
from jax.experimental import pallas as pl
from jax.experimental.pallas import tpu as pltpu

E = 8
TOP_K = 2
D = 768
H = 2 * D
S = 2048
BT = 1024


def _gate_weight_for_expert(xb, Wg, e):
    logits = lax.dot_general(xb, Wg, (((1,), (1,)), ((), ())),
                             preferred_element_type=jnp.float32)
    col = lax.broadcasted_iota(jnp.int32, logits.shape, 1)
    m1 = jnp.max(logits, axis=1, keepdims=True)
    a1 = jnp.min(jnp.where(logits == m1, col, E), axis=1, keepdims=True)
    l2 = jnp.where(col == a1, -jnp.inf, logits)
    m2 = jnp.max(l2, axis=1, keepdims=True)
    a2 = jnp.min(jnp.where(l2 == m2, col, E), axis=1, keepdims=True)
    w1 = 1.0 / (1.0 + jnp.exp(m2 - m1))
    w2 = 1.0 / (1.0 + jnp.exp(m1 - m2))
    we = jnp.where(a1 == e, w1, 0.0) + jnp.where(a2 == e, w2, 0.0)
    return we


def _moe_dense_kernel(xb_ref, Wg_ref, W1_ref, W2_ref, W3_ref, out_ref, acc):
    e = pl.program_id(1)
    xb = xb_ref[...]
    we = _gate_weight_for_expert(xb, Wg_ref[...], e)

    W1e = W1_ref[0]
    W2e = W2_ref[0]
    W3e = W3_ref[0]
    a = lax.dot_general(xb, W1e, (((1,), (1,)), ((), ())),
                        preferred_element_type=jnp.float32)
    xv = lax.dot_general(xb, W2e, (((1,), (1,)), ((), ())),
                         preferred_element_type=jnp.float32)
    res = a * (1.0 / (1.0 + jnp.exp(-a))) * xv
    y = lax.dot_general(res, W3e, (((1,), (1,)), ((), ())),
                        preferred_element_type=jnp.float32)
    contrib = y * we

    @pl.when(e == 0)
    def _():
        acc[...] = contrib

    @pl.when(e > 0)
    def _():
        acc[...] = acc[...] + contrib

    @pl.when(e == E - 1)
    def _():
        out_ref[...] = acc[...]


def kernel(x, Wg, W1, W2, W3):
    x2 = x.reshape(S, D)
    out = pl.pallas_call(
        _moe_dense_kernel,
        grid=(S // BT, E),
        in_specs=[
            pl.BlockSpec((BT, D), lambda t, e: (t, 0)),
            pl.BlockSpec((E, D), lambda t, e: (0, 0)),
            pl.BlockSpec((1, H, D), lambda t, e: (e, 0, 0)),
            pl.BlockSpec((1, H, D), lambda t, e: (e, 0, 0)),
            pl.BlockSpec((1, D, H), lambda t, e: (e, 0, 0)),
        ],
        out_specs=pl.BlockSpec((BT, D), lambda t, e: (t, 0)),
        out_shape=jax.ShapeDtypeStruct((S, D), jnp.float32),
        scratch_shapes=[pltpu.VMEM((BT, D), jnp.float32)],
    )(x2, Wg, W1, W2, W3)
    return out.reshape(1, S, D)
